# baseline (speedup 1.0000x reference)
import functools

import jax
import jax.numpy as jnp
from jax import lax
from jax.experimental import pallas as pl
from jax.experimental.pallas import tpu as pltpu
from jax.experimental.pallas import tpu_sc as plsc

B, N, C, L, K = 2, 2048, 128, 3, 16
BN = B * N
P = BN * K
_BLKN = 128
_BLKP = 256
_BLKD = 512

_f32 = jnp.float32
_i32 = jnp.int32


def _knn_body(xyz_ref, all_ref, out_ref, d2_ref):
    x = xyz_ref[...]
    a = all_ref[...]
    sqx = jnp.sum(x * x, axis=1, keepdims=True)
    sqa = jnp.sum(a * a, axis=1, keepdims=True)
    lane_a = lax.broadcasted_iota(_i32, (N, 128), 1)
    lane_x = lax.broadcasted_iota(_i32, (_BLKN, 128), 1)
    a2 = jnp.where(lane_a == 3, sqa, -2.0 * a)
    x2 = jnp.where(lane_x == 3, 1.0, x)
    dot2 = lax.dot_general(x2, a2, (((1,), (1,)), ((), ())),
                           preferred_element_type=_f32)
    d2 = jnp.maximum(dot2 + sqx, 0.0)

    iota_n = lax.broadcasted_iota(_i32, (_BLKN, N), 1)
    bits = lax.bitcast_convert_type(d2, _i32)
    qk = jnp.bitwise_or(jnp.bitwise_and(bits, ~0x7FF), iota_n)
    d2_ref[...] = qk

    lane8 = lax.broadcasted_iota(_i32, (8, 128), 1)

    def body(i, carry):
        v, acc = carry
        m = jnp.min(v, axis=1, keepdims=True)
        v = jnp.where(v == m, jnp.int32(0x7FFFFFFF), v)
        return v, jnp.where(lane8 == i, m, acc)

    for rg in range(_BLKN // 8):
        sub = d2_ref[pl.ds(rg * 8, 8), :]
        _, acc = lax.fori_loop(0, K + 1, body,
                               (sub, jnp.zeros((8, 128), _i32)))
        out_ref[pl.ds(rg * 8, 8), :] = jnp.bitwise_and(acc, 0x7FF)


def _knn(xyzb):
    return pl.pallas_call(
        _knn_body,
        grid=(N // _BLKN,),
        in_specs=[
            pl.BlockSpec((_BLKN, 128), lambda i: (i, 0)),
            pl.BlockSpec((N, 128), lambda i: (0, 0)),
        ],
        out_specs=pl.BlockSpec((_BLKN, 128), lambda i: (i, 0)),
        out_shape=jax.ShapeDtypeStruct((N, 128), _i32),
        scratch_shapes=[pltpu.VMEM((_BLKN, N), _i32)],
    )(xyzb, xyzb)


def _sc_gather_rows(table, idx, D):
    M = idx.shape[0]
    NW = 32
    CHUNK = 128
    per_w = M // NW
    nchunks = per_w // CHUNK
    NBUF = 2 if D > 256 else 3
    mesh = plsc.VectorSubcoreMesh(core_axis_name="c", subcore_axis_name="s")

    @functools.partial(
        pl.kernel, mesh=mesh,
        out_type=jax.ShapeDtypeStruct((M, D), _f32),
        scratch_types=[pltpu.VMEM((per_w,), _i32)]
        + [pltpu.VMEM((CHUNK, D), _f32) for _ in range(NBUF)]
        + [pltpu.SemaphoreType.DMA for _ in range(2 * NBUF)],
    )
    def k(table_hbm, idx_hbm, out_hbm, idx_v, *bufs):
        rows = bufs[:NBUF]
        gsem = bufs[NBUF:2 * NBUF]
        ssem = bufs[2 * NBUF:]
        wid = lax.axis_index("s") * 2 + lax.axis_index("c")
        base = wid * per_w
        pltpu.sync_copy(idx_hbm.at[pl.ds(base, per_w)], idx_v)

        def gather(c):
            return pltpu.async_copy(
                table_hbm.at[idx_v.at[pl.ds(c * CHUNK, CHUNK)]],
                rows[c % NBUF], gsem[c % NBUF])

        gets = {c: gather(c) for c in range(min(NBUF - 1, nchunks))}
        puts = {}
        for c in range(nchunks):
            nxt = c + NBUF - 1
            if nxt < nchunks:
                if c >= 1:
                    puts[c - 1].wait()
                gets[nxt] = gather(nxt)
            gets[c].wait()
            puts[c] = pltpu.async_copy(
                rows[c % NBUF], out_hbm.at[pl.ds(base + c * CHUNK, CHUNK)],
                ssem[c % NBUF])
        for c in range(max(0, nchunks - NBUF), nchunks):
            puts[c].wait()

    return k(table, idx)


def _embed_body(x_ref, xyzp_ref, w1_ref, b1_ref, w2_ref, b2_ref,
                kvW_ref, kvb_ref, f_out, t_out):
    h = jnp.maximum(
        jnp.dot(x_ref[...], w1_ref[...], preferred_element_type=_f32) + b1_ref[...], 0.0)
    f = jnp.dot(h, w2_ref[...], preferred_element_type=_f32) + b2_ref[...]
    f_out[...] = f
    t_out[:, :256] = jnp.dot(f, kvW_ref[...], preferred_element_type=_f32) + kvb_ref[...]
    t_out[:, 256:] = xyzp_ref[...]


def _embed(x_in, xyzp2, w1, b1, w2, b2, kvW, kvb):
    rows = x_in.shape[0]
    full = lambda a: pl.BlockSpec(a.shape, lambda i: (0,) * a.ndim)
    return pl.pallas_call(
        _embed_body,
        grid=(rows // _BLKD,),
        in_specs=[
            pl.BlockSpec((_BLKD, 8), lambda i: (i, 0)),
            pl.BlockSpec((_BLKD, 128), lambda i: (i, 0)),
            full(w1), full(b1), full(w2), full(b2), full(kvW), full(kvb),
        ],
        out_specs=[
            pl.BlockSpec((_BLKD, 128), lambda i: (i, 0)),
            pl.BlockSpec((_BLKD, 384), lambda i: (i, 0)),
        ],
        out_shape=[
            jax.ShapeDtypeStruct((rows, 128), _f32),
            jax.ShapeDtypeStruct((rows, 384), _f32),
        ],
    )(x_in, xyzp2, w1, b1, w2, b2, kvW, kvb)


def _attn_body(has_kv, f_ref, xi_ref, xnb_ref, kv_ref,
               qW_ref, qb_ref, dW1_ref, db1_ref, dW2_ref, db2_ref,
               gW1_ref, gb1_ref, gW2_ref, gb2_ref, oW_ref, ob_ref,
               *rest):
    if has_kv:
        kvW_ref, kvb_ref, out_ref, kv_out = rest
    else:
        (out_ref,) = rest
    bf = jnp.bfloat16
    mm = lambda a, w: jnp.dot(a.astype(bf), w, preferred_element_type=_f32)
    f = f_ref[...]
    q = mm(f, qW_ref[...]) + qb_ref[...]
    aW = jnp.dot(xi_ref[...], dW1_ref[...], preferred_element_type=_f32)
    bW = jnp.dot(xnb_ref[...], dW1_ref[...], preferred_element_type=_f32)
    pre = aW.reshape(_BLKP, 1, 128) - bW.reshape(_BLKP, K, 128) \
        + db1_ref[...].reshape(1, 1, 128)
    h = jnp.maximum(pre, 0.0).reshape(_BLKP * K, 128)
    pos = mm(h, dW2_ref[...]) + db2_ref[...]
    pos3 = pos.reshape(_BLKP, K, 128)
    kv = kv_ref[...]
    k3 = kv[:, :128].reshape(_BLKP, K, 128)
    v3 = kv[:, 128:].reshape(_BLKP, K, 128)
    attn = (q.reshape(_BLKP, 1, 128) - k3 + pos3).reshape(_BLKP * K, 128)
    hw = jnp.maximum(mm(attn, gW1_ref[...]) + gb1_ref[...], 0.0)
    w = (mm(hw, gW2_ref[...]) + gb2_ref[...]).reshape(_BLKP, K, 128)
    e = jnp.exp(w)
    w = e / jnp.sum(e, axis=1, keepdims=True)
    out = jnp.sum(w * (v3 + pos3), axis=1)
    fn = mm(out, oW_ref[...]) + ob_ref[...] + f
    out_ref[...] = fn
    if has_kv:
        kv_out[...] = mm(fn, kvW_ref[...]) + kvb_ref[...]


def _attn_layer(feats, xyzp2, g0, kvnb, qW, qb, dW1, db1, dW2, db2,
                gW1, gb1, gW2, gb2, oW, ob, kvW=None, kvb=None):
    full = lambda a: pl.BlockSpec(a.shape, lambda i: (0,) * a.ndim)
    has_kv = kvW is not None
    in_specs = [
        pl.BlockSpec((_BLKP, 128), lambda i: (i, 0)),
        pl.BlockSpec((_BLKP, 128), lambda i: (i, 0)),
        pl.BlockSpec((_BLKP * K, 128), lambda i: (i, 2)),
        pl.BlockSpec((_BLKP * K, 256), lambda i: (i, 0)),
        full(qW), full(qb), full(dW1), full(db1), full(dW2), full(db2),
        full(gW1), full(gb1), full(gW2), full(gb2), full(oW), full(ob),
    ]
    args = [feats, xyzp2, g0, kvnb, qW, qb, dW1, db1, dW2, db2,
            gW1, gb1, gW2, gb2, oW, ob]
    rows = feats.shape[0]
    out_specs = [pl.BlockSpec((_BLKP, 128), lambda i: (i, 0))]
    out_shape = [jax.ShapeDtypeStruct((rows, 128), _f32)]
    if has_kv:
        in_specs += [full(kvW), full(kvb)]
        args += [kvW, kvb]
        out_specs.append(pl.BlockSpec((_BLKP, 256), lambda i: (i, 0)))
        out_shape.append(jax.ShapeDtypeStruct((rows, 256), _f32))
    res = pl.pallas_call(
        functools.partial(_attn_body, has_kv),
        grid=(feats.shape[0] // _BLKP,),
        in_specs=in_specs,
        out_specs=out_specs,
        out_shape=out_shape,
    )(*args)
    return res if has_kv else (res[0], None)


def _head_body(f_ref, nrm_ref, w1_ref, b1_ref, w2_ref, b2_ref, w3_ref, b3_ref,
               cw1_ref, cb1_ref, cw2_ref, cb2_ref, ref_out, conf_out):
    f = f_ref[...]
    h1 = jnp.maximum(jnp.dot(f, w1_ref[...], preferred_element_type=_f32) + b1_ref[...], 0.0)
    h2 = jnp.maximum(jnp.dot(h1, w2_ref[...], preferred_element_type=_f32) + b2_ref[...], 0.0)
    delta = jnp.dot(h2, w3_ref[...], preferred_element_type=_f32) + b3_ref[...]
    rn = nrm_ref[...] + delta
    nrm = jnp.sqrt(jnp.sum(rn * rn, axis=1, keepdims=True))
    ref_out[...] = rn / jnp.maximum(nrm, 1e-12)
    c1 = jnp.maximum(jnp.dot(f, cw1_ref[...], preferred_element_type=_f32) + cb1_ref[...], 0.0)
    c2 = jnp.dot(c1, cw2_ref[...], preferred_element_type=_f32) + cb2_ref[...]
    conf_out[...] = 1.0 / (1.0 + jnp.exp(-c2))


def _heads(feats, normalsp, w1, b1, w2, b2, w3, b3, cw1, cb1, cw2, cb2):
    rows = feats.shape[0]
    full = lambda a: pl.BlockSpec(a.shape, lambda i: (0,) * a.ndim)
    return pl.pallas_call(
        _head_body,
        grid=(rows // _BLKD,),
        in_specs=[
            pl.BlockSpec((_BLKD, 128), lambda i: (i, 0)),
            pl.BlockSpec((_BLKD, 128), lambda i: (i, 0)),
            full(w1), full(b1), full(w2), full(b2), full(w3), full(b3),
            full(cw1), full(cb1), full(cw2), full(cb2),
        ],
        out_specs=[
            pl.BlockSpec((_BLKD, 128), lambda i: (i, 0)),
            pl.BlockSpec((_BLKD, 128), lambda i: (i, 0)),
        ],
        out_shape=[
            jax.ShapeDtypeStruct((rows, 128), _f32),
            jax.ShapeDtypeStruct((rows, 128), _f32),
        ],
    )(feats, normalsp, w1, b1, w2, b2, w3, b3, cw1, cb1, cw2, cb2)


def _pad_cols(a, cols):
    return jnp.pad(a, [(0, 0)] * (a.ndim - 1) + [(0, cols - a.shape[-1])])


def _pad2(w, rows, cols):
    return jnp.pad(w, [(0, rows - w.shape[0]), (0, cols - w.shape[1])])


def kernel(xyz, normals, curvature, emb_W1, emb_b1, emb_W2, emb_b2, q_W, q_b,
           k_W, k_b, v_W, v_b, d_W1, d_b1, d_W2, d_b2, g_W1, g_b1, g_W2, g_b2,
           o_W, o_b, nh_W1, nh_b1, nh_W2, nh_b2, nh_W3, nh_b3, ch_W1, ch_b1,
           ch_W2, ch_b2):
    xyzp = _pad_cols(xyz, 128)
    x_in = _pad_cols(jnp.concatenate([xyz, normals, curvature], axis=-1), 8)
    x_in = x_in.reshape(BN, 8)
    embW1 = _pad2(emb_W1, 8, 128)
    bf = jnp.bfloat16
    kvW = jnp.concatenate([k_W, v_W], axis=2)
    kvb = jnp.concatenate([k_b, v_b], axis=1)
    kvWb = kvW.astype(bf)
    qWb, dW2b, gW1b, gW2b, oWb = (w.astype(bf) for w in (q_W, d_W2, g_W1, g_W2, o_W))
    dW1p = jnp.stack([_pad2(d_W1[l], 128, 128) for l in range(L)])
    row = lambda v: v.reshape(1, -1)

    x_in2 = x_in.reshape(B, N, 8)
    xyzb = [xyzp[b] for b in range(B)]
    feats, kv, idxg = [None] * B, [None] * B, [None] * B
    for b in range(B):
        knn_out = _knn(xyzb[b])
        idxg[b] = knn_out[:, 1:K + 1].reshape(N * K)
        feats[b], kv[b] = _embed(x_in2[b], xyzb[b], embW1, row(emb_b1),
                                 emb_W2, row(emb_b2), kvW[0], row(kvb[0]))

    g0 = [None] * B
    for l in range(L):
        kvnb = [None] * B
        for b in range(B):
            kvnb[b] = _sc_gather_rows(kv[b], idxg[b], kv[b].shape[1])
            if l == 0:
                g0[b] = kvnb[b]
        nxt = (kvWb[l + 1], row(kvb[l + 1])) if l + 1 < L else (None, None)
        for b in range(B):
            feats[b], kv[b] = _attn_layer(
                feats[b], xyzb[b], g0[b], kvnb[b],
                qWb[l], row(q_b[l]), dW1p[l], row(d_b1[l]),
                dW2b[l], row(d_b2[l]), gW1b[l], row(g_b1[l]),
                gW2b[l], row(g_b2[l]), oWb[l], row(o_b[l]),
                kvW=nxt[0], kvb=nxt[1])

    normalsp = _pad_cols(normals, 128)
    nhW2p = _pad2(nh_W2, 128, 128)
    nhW3p = _pad2(nh_W3, 128, 128)
    chW1p = _pad2(ch_W1, 128, 128)
    chW2p = _pad2(ch_W2, 128, 128)
    refined, conf = [None] * B, [None] * B
    for b in range(B):
        rb, cb = _heads(
            feats[b], normalsp[b], nh_W1, row(nh_b1), nhW2p,
            row(_pad_cols(nh_b2, 128)), nhW3p, row(_pad_cols(nh_b3, 128)),
            chW1p, row(_pad_cols(ch_b1, 128)), chW2p, row(_pad_cols(ch_b2, 128)))
        refined[b], conf[b] = rb[:, :3], cb[:, :1]

    return jnp.stack(refined), jnp.stack(conf)


# speedup vs baseline: 3.5178x; 3.5178x over previous
import functools

import jax
import jax.numpy as jnp
from jax import lax
from jax.experimental import pallas as pl
from jax.experimental.pallas import tpu as pltpu
from jax.experimental.pallas import tpu_sc as plsc

B, N, C, L, K = 2, 2048, 128, 3, 16
BN = B * N
P = BN * K
_BLKN = 128
_BLKP = 256
_BLKD = 512

_f32 = jnp.float32
_i32 = jnp.int32


def _knn_body(xyz_ref, all_ref, out_ref, d2_ref):
    x = xyz_ref[...]
    a = all_ref[...]
    sqx = jnp.sum(x * x, axis=1, keepdims=True)
    sqa = jnp.sum(a * a, axis=1, keepdims=True)
    lane_a = lax.broadcasted_iota(_i32, (N, 128), 1)
    lane_x = lax.broadcasted_iota(_i32, (_BLKN, 128), 1)
    a2 = jnp.where(lane_a == 3, sqa, -2.0 * a)
    x2 = jnp.where(lane_x == 3, 1.0, x)
    dot2 = lax.dot_general(x2, a2, (((1,), (1,)), ((), ())),
                           preferred_element_type=_f32)
    d2 = jnp.maximum(dot2 + sqx, 0.0)

    iota_n = lax.broadcasted_iota(_i32, (_BLKN, N), 1)
    bits = lax.bitcast_convert_type(d2, _i32)
    d2_ref[...] = jnp.bitwise_or(jnp.bitwise_and(bits, ~0x7FF), iota_n)

    def body(i, acc):
        v = d2_ref[...]
        m = jnp.min(v, axis=1, keepdims=True)
        d2_ref[...] = jnp.where(v == m, jnp.int32(0x7FFFFFFF), v)
        return jnp.where(lane_x == i, m, acc)

    acc = lax.fori_loop(0, K + 1, body, jnp.zeros((_BLKN, 128), _i32))
    out_ref[...] = jnp.bitwise_and(acc, 0x7FF)


def _knn(xyzb):
    return pl.pallas_call(
        _knn_body,
        grid=(N // _BLKN,),
        in_specs=[
            pl.BlockSpec((_BLKN, 128), lambda i: (i, 0)),
            pl.BlockSpec((N, 128), lambda i: (0, 0)),
        ],
        out_specs=pl.BlockSpec((_BLKN, 128), lambda i: (i, 0)),
        out_shape=jax.ShapeDtypeStruct((N, 128), _i32),
        scratch_shapes=[pltpu.VMEM((_BLKN, N), _i32)],
    )(xyzb, xyzb)


def _sc_gather_rows(table, idx, D):
    M = idx.shape[0]
    NW = 32
    CHUNK = 128
    per_w = M // NW
    nchunks = per_w // CHUNK
    NBUF = 2 if D > 256 else 3
    mesh = plsc.VectorSubcoreMesh(core_axis_name="c", subcore_axis_name="s")

    @functools.partial(
        pl.kernel, mesh=mesh,
        out_type=jax.ShapeDtypeStruct((M, D), _f32),
        scratch_types=[pltpu.VMEM((per_w,), _i32)]
        + [pltpu.VMEM((CHUNK, D), _f32) for _ in range(NBUF)]
        + [pltpu.SemaphoreType.DMA for _ in range(2 * NBUF)],
    )
    def k(table_hbm, idx_hbm, out_hbm, idx_v, *bufs):
        rows = bufs[:NBUF]
        gsem = bufs[NBUF:2 * NBUF]
        ssem = bufs[2 * NBUF:]
        wid = lax.axis_index("s") * 2 + lax.axis_index("c")
        base = wid * per_w
        pltpu.sync_copy(idx_hbm.at[pl.ds(base, per_w)], idx_v)

        def gather(c):
            return pltpu.async_copy(
                table_hbm.at[idx_v.at[pl.ds(c * CHUNK, CHUNK)]],
                rows[c % NBUF], gsem[c % NBUF])

        gets = {c: gather(c) for c in range(min(NBUF - 1, nchunks))}
        puts = {}
        for c in range(nchunks):
            nxt = c + NBUF - 1
            if nxt < nchunks:
                if c >= 1:
                    puts[c - 1].wait()
                gets[nxt] = gather(nxt)
            gets[c].wait()
            puts[c] = pltpu.async_copy(
                rows[c % NBUF], out_hbm.at[pl.ds(base + c * CHUNK, CHUNK)],
                ssem[c % NBUF])
        for c in range(max(0, nchunks - NBUF), nchunks):
            puts[c].wait()

    return k(table, idx)


def _embed_body(x_ref, xyzp_ref, w1_ref, b1_ref, w2_ref, b2_ref,
                kvW_ref, kvb_ref, f_out, t_out):
    h = jnp.maximum(
        jnp.dot(x_ref[...], w1_ref[...], preferred_element_type=_f32) + b1_ref[...], 0.0)
    f = jnp.dot(h, w2_ref[...], preferred_element_type=_f32) + b2_ref[...]
    f_out[...] = f
    t_out[:, :256] = jnp.dot(f, kvW_ref[...], preferred_element_type=_f32) + kvb_ref[...]
    t_out[:, 256:] = xyzp_ref[...]


def _embed(x_in, xyzp2, w1, b1, w2, b2, kvW, kvb):
    rows = x_in.shape[0]
    full = lambda a: pl.BlockSpec(a.shape, lambda i: (0,) * a.ndim)
    return pl.pallas_call(
        _embed_body,
        grid=(rows // _BLKD,),
        in_specs=[
            pl.BlockSpec((_BLKD, 8), lambda i: (i, 0)),
            pl.BlockSpec((_BLKD, 128), lambda i: (i, 0)),
            full(w1), full(b1), full(w2), full(b2), full(kvW), full(kvb),
        ],
        out_specs=[
            pl.BlockSpec((_BLKD, 128), lambda i: (i, 0)),
            pl.BlockSpec((_BLKD, 384), lambda i: (i, 0)),
        ],
        out_shape=[
            jax.ShapeDtypeStruct((rows, 128), _f32),
            jax.ShapeDtypeStruct((rows, 384), _f32),
        ],
    )(x_in, xyzp2, w1, b1, w2, b2, kvW, kvb)


def _attn_body(has_kv, f_ref, xi_ref, xnb_ref, kv_ref,
               qW_ref, qb_ref, dW1_ref, db1_ref, dW2_ref, db2_ref,
               gW1_ref, gb1_ref, gW2_ref, gb2_ref, oW_ref, ob_ref,
               *rest):
    if has_kv:
        kvW_ref, kvb_ref, out_ref, kv_out = rest
    else:
        (out_ref,) = rest
    bf = jnp.bfloat16
    mm = lambda a, w: jnp.dot(a.astype(bf), w, preferred_element_type=_f32)
    f = f_ref[...]
    q = mm(f, qW_ref[...]) + qb_ref[...]
    aW = jnp.dot(xi_ref[...], dW1_ref[...], preferred_element_type=_f32)
    bW = jnp.dot(xnb_ref[...], dW1_ref[...], preferred_element_type=_f32)
    pre = aW.reshape(_BLKP, 1, 128) - bW.reshape(_BLKP, K, 128) \
        + db1_ref[...].reshape(1, 1, 128)
    h = jnp.maximum(pre, 0.0).reshape(_BLKP * K, 128)
    pos = mm(h, dW2_ref[...]) + db2_ref[...]
    pos3 = pos.reshape(_BLKP, K, 128)
    kv = kv_ref[...]
    k3 = kv[:, :128].reshape(_BLKP, K, 128)
    v3 = kv[:, 128:].reshape(_BLKP, K, 128)
    attn = (q.reshape(_BLKP, 1, 128) - k3 + pos3).reshape(_BLKP * K, 128)
    hw = jnp.maximum(mm(attn, gW1_ref[...]) + gb1_ref[...], 0.0)
    w = (mm(hw, gW2_ref[...]) + gb2_ref[...]).reshape(_BLKP, K, 128)
    e = jnp.exp(w)
    w = e / jnp.sum(e, axis=1, keepdims=True)
    out = jnp.sum(w * (v3 + pos3), axis=1)
    fn = mm(out, oW_ref[...]) + ob_ref[...] + f
    out_ref[...] = fn
    if has_kv:
        kv_out[...] = mm(fn, kvW_ref[...]) + kvb_ref[...]


def _attn_layer(feats, xyzp2, g0, kvnb, qW, qb, dW1, db1, dW2, db2,
                gW1, gb1, gW2, gb2, oW, ob, kvW=None, kvb=None):
    full = lambda a: pl.BlockSpec(a.shape, lambda i: (0,) * a.ndim)
    has_kv = kvW is not None
    in_specs = [
        pl.BlockSpec((_BLKP, 128), lambda i: (i, 0)),
        pl.BlockSpec((_BLKP, 128), lambda i: (i, 0)),
        pl.BlockSpec((_BLKP * K, 128), lambda i: (i, 2)),
        pl.BlockSpec((_BLKP * K, 256), lambda i: (i, 0)),
        full(qW), full(qb), full(dW1), full(db1), full(dW2), full(db2),
        full(gW1), full(gb1), full(gW2), full(gb2), full(oW), full(ob),
    ]
    args = [feats, xyzp2, g0, kvnb, qW, qb, dW1, db1, dW2, db2,
            gW1, gb1, gW2, gb2, oW, ob]
    rows = feats.shape[0]
    out_specs = [pl.BlockSpec((_BLKP, 128), lambda i: (i, 0))]
    out_shape = [jax.ShapeDtypeStruct((rows, 128), _f32)]
    if has_kv:
        in_specs += [full(kvW), full(kvb)]
        args += [kvW, kvb]
        out_specs.append(pl.BlockSpec((_BLKP, 256), lambda i: (i, 0)))
        out_shape.append(jax.ShapeDtypeStruct((rows, 256), _f32))
    res = pl.pallas_call(
        functools.partial(_attn_body, has_kv),
        grid=(feats.shape[0] // _BLKP,),
        in_specs=in_specs,
        out_specs=out_specs,
        out_shape=out_shape,
    )(*args)
    return res if has_kv else (res[0], None)


def _head_body(f_ref, nrm_ref, w1_ref, b1_ref, w2_ref, b2_ref, w3_ref, b3_ref,
               cw1_ref, cb1_ref, cw2_ref, cb2_ref, ref_out, conf_out):
    f = f_ref[...]
    h1 = jnp.maximum(jnp.dot(f, w1_ref[...], preferred_element_type=_f32) + b1_ref[...], 0.0)
    h2 = jnp.maximum(jnp.dot(h1, w2_ref[...], preferred_element_type=_f32) + b2_ref[...], 0.0)
    delta = jnp.dot(h2, w3_ref[...], preferred_element_type=_f32) + b3_ref[...]
    rn = nrm_ref[...] + delta
    nrm = jnp.sqrt(jnp.sum(rn * rn, axis=1, keepdims=True))
    ref_out[...] = rn / jnp.maximum(nrm, 1e-12)
    c1 = jnp.maximum(jnp.dot(f, cw1_ref[...], preferred_element_type=_f32) + cb1_ref[...], 0.0)
    c2 = jnp.dot(c1, cw2_ref[...], preferred_element_type=_f32) + cb2_ref[...]
    conf_out[...] = 1.0 / (1.0 + jnp.exp(-c2))


def _heads(feats, normalsp, w1, b1, w2, b2, w3, b3, cw1, cb1, cw2, cb2):
    rows = feats.shape[0]
    full = lambda a: pl.BlockSpec(a.shape, lambda i: (0,) * a.ndim)
    return pl.pallas_call(
        _head_body,
        grid=(rows // _BLKD,),
        in_specs=[
            pl.BlockSpec((_BLKD, 128), lambda i: (i, 0)),
            pl.BlockSpec((_BLKD, 128), lambda i: (i, 0)),
            full(w1), full(b1), full(w2), full(b2), full(w3), full(b3),
            full(cw1), full(cb1), full(cw2), full(cb2),
        ],
        out_specs=[
            pl.BlockSpec((_BLKD, 128), lambda i: (i, 0)),
            pl.BlockSpec((_BLKD, 128), lambda i: (i, 0)),
        ],
        out_shape=[
            jax.ShapeDtypeStruct((rows, 128), _f32),
            jax.ShapeDtypeStruct((rows, 128), _f32),
        ],
    )(feats, normalsp, w1, b1, w2, b2, w3, b3, cw1, cb1, cw2, cb2)


def _pad_cols(a, cols):
    return jnp.pad(a, [(0, 0)] * (a.ndim - 1) + [(0, cols - a.shape[-1])])


def _pad2(w, rows, cols):
    return jnp.pad(w, [(0, rows - w.shape[0]), (0, cols - w.shape[1])])


def kernel(xyz, normals, curvature, emb_W1, emb_b1, emb_W2, emb_b2, q_W, q_b,
           k_W, k_b, v_W, v_b, d_W1, d_b1, d_W2, d_b2, g_W1, g_b1, g_W2, g_b2,
           o_W, o_b, nh_W1, nh_b1, nh_W2, nh_b2, nh_W3, nh_b3, ch_W1, ch_b1,
           ch_W2, ch_b2):
    xyzp = _pad_cols(xyz, 128)
    x_in = _pad_cols(jnp.concatenate([xyz, normals, curvature], axis=-1), 8)
    x_in = x_in.reshape(BN, 8)
    embW1 = _pad2(emb_W1, 8, 128)
    bf = jnp.bfloat16
    kvW = jnp.concatenate([k_W, v_W], axis=2)
    kvb = jnp.concatenate([k_b, v_b], axis=1)
    kvWb = kvW.astype(bf)
    qWb, dW2b, gW1b, gW2b, oWb = (w.astype(bf) for w in (q_W, d_W2, g_W1, g_W2, o_W))
    dW1p = jnp.stack([_pad2(d_W1[l], 128, 128) for l in range(L)])
    row = lambda v: v.reshape(1, -1)

    x_in2 = x_in.reshape(B, N, 8)
    xyzb = [xyzp[b] for b in range(B)]
    feats, kv, idxg = [None] * B, [None] * B, [None] * B
    for b in range(B):
        knn_out = _knn(xyzb[b])
        idxg[b] = knn_out[:, 1:K + 1].reshape(N * K)
        feats[b], kv[b] = _embed(x_in2[b], xyzb[b], embW1, row(emb_b1),
                                 emb_W2, row(emb_b2), kvW[0], row(kvb[0]))

    g0 = [None] * B
    for l in range(L):
        kvnb = [None] * B
        for b in range(B):
            kvnb[b] = _sc_gather_rows(kv[b], idxg[b], kv[b].shape[1])
            if l == 0:
                g0[b] = kvnb[b]
        nxt = (kvWb[l + 1], row(kvb[l + 1])) if l + 1 < L else (None, None)
        for b in range(B):
            feats[b], kv[b] = _attn_layer(
                feats[b], xyzb[b], g0[b], kvnb[b],
                qWb[l], row(q_b[l]), dW1p[l], row(d_b1[l]),
                dW2b[l], row(d_b2[l]), gW1b[l], row(g_b1[l]),
                gW2b[l], row(g_b2[l]), oWb[l], row(o_b[l]),
                kvW=nxt[0], kvb=nxt[1])

    normalsp = _pad_cols(normals, 128)
    nhW2p = _pad2(nh_W2, 128, 128)
    nhW3p = _pad2(nh_W3, 128, 128)
    chW1p = _pad2(ch_W1, 128, 128)
    chW2p = _pad2(ch_W2, 128, 128)
    refined, conf = [None] * B, [None] * B
    for b in range(B):
        rb, cb = _heads(
            feats[b], normalsp[b], nh_W1, row(nh_b1), nhW2p,
            row(_pad_cols(nh_b2, 128)), nhW3p, row(_pad_cols(nh_b3, 128)),
            chW1p, row(_pad_cols(ch_b1, 128)), chW2p, row(_pad_cols(ch_b2, 128)))
        refined[b], conf[b] = rb[:, :3], cb[:, :1]

    return jnp.stack(refined), jnp.stack(conf)


# speedup vs baseline: 3.5246x; 1.0019x over previous
import functools

import jax
import jax.numpy as jnp
from jax import lax
from jax.experimental import pallas as pl
from jax.experimental.pallas import tpu as pltpu
from jax.experimental.pallas import tpu_sc as plsc

B, N, C, L, K = 2, 2048, 128, 3, 16
BN = B * N
P = BN * K
_BLKN = 128
_BLKP = 256
_BLKD = 512

_f32 = jnp.float32
_i32 = jnp.int32


def _knn_body(xyz_ref, all_ref, out_ref, d2_ref):
    x = xyz_ref[...]
    a = all_ref[...]
    sqx = jnp.sum(x * x, axis=1, keepdims=True)
    sqa = jnp.sum(a * a, axis=1, keepdims=True)
    lane_a = lax.broadcasted_iota(_i32, (N, 128), 1)
    lane_x = lax.broadcasted_iota(_i32, (_BLKN, 128), 1)
    a2 = jnp.where(lane_a == 3, sqa, -2.0 * a)
    x2 = jnp.where(lane_x == 3, 1.0, x)
    dot2 = lax.dot_general(x2, a2, (((1,), (1,)), ((), ())),
                           preferred_element_type=_f32)
    d2 = jnp.maximum(dot2 + sqx, 0.0)

    iota_n = lax.broadcasted_iota(_i32, (_BLKN, N), 1)
    bits = lax.bitcast_convert_type(d2, _i32)
    d2_ref[...] = jnp.bitwise_or(jnp.bitwise_and(bits, ~0x7FF), iota_n)

    def body(i, acc):
        v = d2_ref[...]
        m = jnp.min(v, axis=1, keepdims=True)
        d2_ref[...] = jnp.where(v == m, jnp.int32(0x7FFFFFFF), v)
        return jnp.where(lane_x == i, m, acc)

    acc = lax.fori_loop(0, K + 1, body, jnp.zeros((_BLKN, 128), _i32))
    out_ref[...] = jnp.bitwise_and(acc, 0x7FF)


def _knn(xyzb):
    return pl.pallas_call(
        _knn_body,
        grid=(N // _BLKN,),
        in_specs=[
            pl.BlockSpec((_BLKN, 128), lambda i: (i, 0)),
            pl.BlockSpec((N, 128), lambda i: (0, 0)),
        ],
        out_specs=pl.BlockSpec((_BLKN, 128), lambda i: (i, 0)),
        out_shape=jax.ShapeDtypeStruct((N, 128), _i32),
        scratch_shapes=[pltpu.VMEM((_BLKN, N), _i32)],
    )(xyzb, xyzb)


def _sc_gather_rows(table, idx, D):
    M = idx.shape[0]
    NW = 32
    CHUNK = 128
    per_w = M // NW
    nchunks = per_w // CHUNK
    NBUF = 2 if D > 256 else 3
    mesh = plsc.VectorSubcoreMesh(core_axis_name="c", subcore_axis_name="s")

    @functools.partial(
        pl.kernel, mesh=mesh,
        out_type=jax.ShapeDtypeStruct((M, D), _f32),
        scratch_types=[pltpu.VMEM((per_w,), _i32)]
        + [pltpu.VMEM((CHUNK, D), _f32) for _ in range(NBUF)]
        + [pltpu.SemaphoreType.DMA for _ in range(2 * NBUF)],
    )
    def k(table_hbm, idx_hbm, out_hbm, idx_v, *bufs):
        rows = bufs[:NBUF]
        gsem = bufs[NBUF:2 * NBUF]
        ssem = bufs[2 * NBUF:]
        wid = lax.axis_index("s") * 2 + lax.axis_index("c")
        base = wid * per_w
        pltpu.sync_copy(idx_hbm.at[pl.ds(base, per_w)], idx_v)

        def gather(c):
            return pltpu.async_copy(
                table_hbm.at[idx_v.at[pl.ds(c * CHUNK, CHUNK)]],
                rows[c % NBUF], gsem[c % NBUF])

        gets = {c: gather(c) for c in range(min(NBUF - 1, nchunks))}
        puts = {}
        for c in range(nchunks):
            nxt = c + NBUF - 1
            if nxt < nchunks:
                if c >= 1:
                    puts[c - 1].wait()
                gets[nxt] = gather(nxt)
            gets[c].wait()
            puts[c] = pltpu.async_copy(
                rows[c % NBUF], out_hbm.at[pl.ds(base + c * CHUNK, CHUNK)],
                ssem[c % NBUF])
        for c in range(max(0, nchunks - NBUF), nchunks):
            puts[c].wait()

    return k(table, idx)


def _embed_body(x_ref, xyzp_ref, w1_ref, b1_ref, w2_ref, b2_ref,
                kvW_ref, kvb_ref, f_out, t_out):
    h = jnp.maximum(
        jnp.dot(x_ref[...], w1_ref[...], preferred_element_type=_f32) + b1_ref[...], 0.0)
    f = jnp.dot(h, w2_ref[...], preferred_element_type=_f32) + b2_ref[...]
    f_out[...] = f
    t_out[:, :256] = jnp.dot(f, kvW_ref[...], preferred_element_type=_f32) + kvb_ref[...]
    t_out[:, 256:] = xyzp_ref[...]


def _embed(x_in, xyzp2, w1, b1, w2, b2, kvW, kvb):
    rows = x_in.shape[0]
    full = lambda a: pl.BlockSpec(a.shape, lambda i: (0,) * a.ndim)
    return pl.pallas_call(
        _embed_body,
        grid=(rows // _BLKD,),
        in_specs=[
            pl.BlockSpec((_BLKD, 8), lambda i: (i, 0)),
            pl.BlockSpec((_BLKD, 128), lambda i: (i, 0)),
            full(w1), full(b1), full(w2), full(b2), full(kvW), full(kvb),
        ],
        out_specs=[
            pl.BlockSpec((_BLKD, 128), lambda i: (i, 0)),
            pl.BlockSpec((_BLKD, 384), lambda i: (i, 0)),
        ],
        out_shape=[
            jax.ShapeDtypeStruct((rows, 128), _f32),
            jax.ShapeDtypeStruct((rows, 384), _f32),
        ],
    )(x_in, xyzp2, w1, b1, w2, b2, kvW, kvb)


def _attn_body(has_kv, f_ref, xi_ref, xnb_ref, kv_ref,
               qW_ref, qb_ref, dW1_ref, db1_ref, dW2_ref, db2_ref,
               gW1_ref, gb1_ref, gW2_ref, gb2_ref, oW_ref, ob_ref,
               *rest):
    if has_kv:
        kvW_ref, kvb_ref, out_ref, kv_out = rest
    else:
        (out_ref,) = rest
    bf = jnp.bfloat16
    mm = lambda a, w: jnp.dot(a.astype(bf), w, preferred_element_type=_f32)
    f = f_ref[...]
    q = mm(f, qW_ref[...]) + qb_ref[...]
    aW = jnp.dot(xi_ref[...], dW1_ref[...], preferred_element_type=_f32)
    bW = jnp.dot(xnb_ref[...], dW1_ref[...], preferred_element_type=_f32)
    pre = aW.reshape(_BLKP, 1, 128) - bW.reshape(_BLKP, K, 128) \
        + db1_ref[...].reshape(1, 1, 128)
    h = jnp.maximum(pre, 0.0).reshape(_BLKP * K, 128)
    pos = mm(h, dW2_ref[...]) + db2_ref[...]
    pos3 = pos.reshape(_BLKP, K, 128)
    kv = kv_ref[...]
    k3 = kv[:, :128].reshape(_BLKP, K, 128)
    v3 = kv[:, 128:].reshape(_BLKP, K, 128)
    attn = (q.reshape(_BLKP, 1, 128) - k3 + pos3).reshape(_BLKP * K, 128)
    hw = jnp.maximum(mm(attn, gW1_ref[...]) + gb1_ref[...], 0.0)
    w = (mm(hw, gW2_ref[...]) + gb2_ref[...]).reshape(_BLKP, K, 128)
    e = jnp.exp(w)
    out = jnp.sum(e * (v3 + pos3), axis=1) / jnp.sum(e, axis=1)
    fn = mm(out, oW_ref[...]) + ob_ref[...] + f
    out_ref[...] = fn
    if has_kv:
        kv_out[...] = mm(fn, kvW_ref[...]) + kvb_ref[...]


def _attn_layer(feats, xyzp2, g0, kvnb, qW, qb, dW1, db1, dW2, db2,
                gW1, gb1, gW2, gb2, oW, ob, kvW=None, kvb=None):
    full = lambda a: pl.BlockSpec(a.shape, lambda i: (0,) * a.ndim)
    has_kv = kvW is not None
    in_specs = [
        pl.BlockSpec((_BLKP, 128), lambda i: (i, 0)),
        pl.BlockSpec((_BLKP, 128), lambda i: (i, 0)),
        pl.BlockSpec((_BLKP * K, 128), lambda i: (i, 2)),
        pl.BlockSpec((_BLKP * K, 256), lambda i: (i, 0)),
        full(qW), full(qb), full(dW1), full(db1), full(dW2), full(db2),
        full(gW1), full(gb1), full(gW2), full(gb2), full(oW), full(ob),
    ]
    args = [feats, xyzp2, g0, kvnb, qW, qb, dW1, db1, dW2, db2,
            gW1, gb1, gW2, gb2, oW, ob]
    rows = feats.shape[0]
    out_specs = [pl.BlockSpec((_BLKP, 128), lambda i: (i, 0))]
    out_shape = [jax.ShapeDtypeStruct((rows, 128), _f32)]
    if has_kv:
        in_specs += [full(kvW), full(kvb)]
        args += [kvW, kvb]
        out_specs.append(pl.BlockSpec((_BLKP, 256), lambda i: (i, 0)))
        out_shape.append(jax.ShapeDtypeStruct((rows, 256), _f32))
    res = pl.pallas_call(
        functools.partial(_attn_body, has_kv),
        grid=(feats.shape[0] // _BLKP,),
        in_specs=in_specs,
        out_specs=out_specs,
        out_shape=out_shape,
    )(*args)
    return res if has_kv else (res[0], None)


def _head_body(f_ref, nrm_ref, w1_ref, b1_ref, w2_ref, b2_ref, w3_ref, b3_ref,
               cw1_ref, cb1_ref, cw2_ref, cb2_ref, ref_out, conf_out):
    f = f_ref[...]
    h1 = jnp.maximum(jnp.dot(f, w1_ref[...], preferred_element_type=_f32) + b1_ref[...], 0.0)
    h2 = jnp.maximum(jnp.dot(h1, w2_ref[...], preferred_element_type=_f32) + b2_ref[...], 0.0)
    delta = jnp.dot(h2, w3_ref[...], preferred_element_type=_f32) + b3_ref[...]
    rn = nrm_ref[...] + delta
    nrm = jnp.sqrt(jnp.sum(rn * rn, axis=1, keepdims=True))
    ref_out[...] = rn / jnp.maximum(nrm, 1e-12)
    c1 = jnp.maximum(jnp.dot(f, cw1_ref[...], preferred_element_type=_f32) + cb1_ref[...], 0.0)
    c2 = jnp.dot(c1, cw2_ref[...], preferred_element_type=_f32) + cb2_ref[...]
    conf_out[...] = 1.0 / (1.0 + jnp.exp(-c2))


def _heads(feats, normalsp, w1, b1, w2, b2, w3, b3, cw1, cb1, cw2, cb2):
    rows = feats.shape[0]
    full = lambda a: pl.BlockSpec(a.shape, lambda i: (0,) * a.ndim)
    return pl.pallas_call(
        _head_body,
        grid=(rows // _BLKD,),
        in_specs=[
            pl.BlockSpec((_BLKD, 128), lambda i: (i, 0)),
            pl.BlockSpec((_BLKD, 128), lambda i: (i, 0)),
            full(w1), full(b1), full(w2), full(b2), full(w3), full(b3),
            full(cw1), full(cb1), full(cw2), full(cb2),
        ],
        out_specs=[
            pl.BlockSpec((_BLKD, 128), lambda i: (i, 0)),
            pl.BlockSpec((_BLKD, 128), lambda i: (i, 0)),
        ],
        out_shape=[
            jax.ShapeDtypeStruct((rows, 128), _f32),
            jax.ShapeDtypeStruct((rows, 128), _f32),
        ],
    )(feats, normalsp, w1, b1, w2, b2, w3, b3, cw1, cb1, cw2, cb2)


def _pad_cols(a, cols):
    return jnp.pad(a, [(0, 0)] * (a.ndim - 1) + [(0, cols - a.shape[-1])])


def _pad2(w, rows, cols):
    return jnp.pad(w, [(0, rows - w.shape[0]), (0, cols - w.shape[1])])


def kernel(xyz, normals, curvature, emb_W1, emb_b1, emb_W2, emb_b2, q_W, q_b,
           k_W, k_b, v_W, v_b, d_W1, d_b1, d_W2, d_b2, g_W1, g_b1, g_W2, g_b2,
           o_W, o_b, nh_W1, nh_b1, nh_W2, nh_b2, nh_W3, nh_b3, ch_W1, ch_b1,
           ch_W2, ch_b2):
    xyzp = _pad_cols(xyz, 128)
    x_in = _pad_cols(jnp.concatenate([xyz, normals, curvature], axis=-1), 8)
    x_in = x_in.reshape(BN, 8)
    embW1 = _pad2(emb_W1, 8, 128)
    bf = jnp.bfloat16
    kvW = jnp.concatenate([k_W, v_W], axis=2)
    kvb = jnp.concatenate([k_b, v_b], axis=1)
    kvWb = kvW.astype(bf)
    qWb, dW2b, gW1b, gW2b, oWb = (w.astype(bf) for w in (q_W, d_W2, g_W1, g_W2, o_W))
    dW1p = jnp.stack([_pad2(d_W1[l], 128, 128) for l in range(L)])
    row = lambda v: v.reshape(1, -1)

    x_in2 = x_in.reshape(B, N, 8)
    xyzb = [xyzp[b] for b in range(B)]
    feats, kv, idxg = [None] * B, [None] * B, [None] * B
    for b in range(B):
        knn_out = _knn(xyzb[b])
        idxg[b] = knn_out[:, 1:K + 1].reshape(N * K)
        feats[b], kv[b] = _embed(x_in2[b], xyzb[b], embW1, row(emb_b1),
                                 emb_W2, row(emb_b2), kvW[0], row(kvb[0]))

    g0 = [None] * B
    for l in range(L):
        kvnb = [None] * B
        for b in range(B):
            kvnb[b] = _sc_gather_rows(kv[b], idxg[b], kv[b].shape[1])
            if l == 0:
                g0[b] = kvnb[b]
        nxt = (kvWb[l + 1], row(kvb[l + 1])) if l + 1 < L else (None, None)
        for b in range(B):
            feats[b], kv[b] = _attn_layer(
                feats[b], xyzb[b], g0[b], kvnb[b],
                qWb[l], row(q_b[l]), dW1p[l], row(d_b1[l]),
                dW2b[l], row(d_b2[l]), gW1b[l], row(g_b1[l]),
                gW2b[l], row(g_b2[l]), oWb[l], row(o_b[l]),
                kvW=nxt[0], kvb=nxt[1])

    normalsp = _pad_cols(normals, 128)
    nhW2p = _pad2(nh_W2, 128, 128)
    nhW3p = _pad2(nh_W3, 128, 128)
    chW1p = _pad2(ch_W1, 128, 128)
    chW2p = _pad2(ch_W2, 128, 128)
    refined, conf = [None] * B, [None] * B
    for b in range(B):
        rb, cb = _heads(
            feats[b], normalsp[b], nh_W1, row(nh_b1), nhW2p,
            row(_pad_cols(nh_b2, 128)), nhW3p, row(_pad_cols(nh_b3, 128)),
            chW1p, row(_pad_cols(ch_b1, 128)), chW2p, row(_pad_cols(ch_b2, 128)))
        refined[b], conf[b] = rb[:, :3], cb[:, :1]

    return jnp.stack(refined), jnp.stack(conf)
